# baseline (speedup 1.0000x reference)
import functools

import jax
import jax.numpy as jnp
from jax import lax
from jax.experimental import pallas as pl
from jax.experimental.pallas import tpu as pltpu
from jax.experimental.pallas import tpu_sc as plsc

N = 10000
E = 160000
D = 256
R = 8
H = 128
WPR = D // 2
WH = H // 2
NC = 2
NT = 16
CH = 40
EPT = E // NT
NCH = EPT // CH
NBUF = 4
NFB = 2
BN = 1000
NB = N // BN
ZR = 40
NZ = N // ZR


def _mm_body(feat_ref, w_ref, t_ref):
    y = jnp.dot(feat_ref[...], w_ref[0], preferred_element_type=jnp.float32)

    def _pack(a, b):
        a16 = lax.bitcast_convert_type(a.astype(jnp.bfloat16), jnp.uint16)
        b16 = lax.bitcast_convert_type(b.astype(jnp.bfloat16), jnp.uint16)
        return a16.astype(jnp.int32) | (b16.astype(jnp.int32) << 16)

    t_ref[...] = jnp.concatenate(
        [_pack(y[:, c * H:c * H + WH], y[:, c * H + WH:(c + 1) * H])
         for c in range(NC)], axis=1)


def _transform(feat, weight):
    return pl.pallas_call(
        _mm_body,
        grid=(NB, R),
        in_specs=[
            pl.BlockSpec((BN, D), lambda i, r: (i, 0)),
            pl.BlockSpec((1, D, D), lambda i, r: (r, 0, 0)),
        ],
        out_specs=pl.BlockSpec((BN, WPR), lambda i, r: (r * NB + i, 0)),
        out_shape=jax.ShapeDtypeStruct((R * N, WPR), jnp.int32),
    )(feat, weight)


def _sc_body(t_hbm, src_hbm, et_hbm, dst_hbm, out_hbm,
             accum, src_v, et_v, dst_v, key_v, sdst_v, rows_v, full_v,
             *sems):
    c = lax.axis_index("c")
    s = lax.axis_index("s")
    sem_i = sems[0:NBUF]
    sem_g = sems[NBUF:2 * NBUF]
    sem_s = sems[2 * NBUF:2 * NBUF + NFB]

    def _zrow(j, carry):
        for k in range(H // 16):
            full_v[0, j, pl.ds(k * 16, 16)] = jnp.zeros((16,), jnp.float32)
        return carry
    lax.fori_loop(0, ZR, _zrow, 0)
    nzc = (NZ - s + NT - 1) // NT

    def _zchunk(i, carry):
        pltpu.sync_copy(full_v.at[0], accum.at[pl.ds((s + i * NT) * ZR, ZR)])
        return carry
    lax.fori_loop(0, nzc, _zchunk, 0)
    plsc.subcore_barrier()

    ebase = s * EPT

    def _idx_start(g, b):
        e0 = ebase + g * CH
        pltpu.make_async_copy(src_hbm.at[pl.ds(e0, CH)], src_v.at[b], sem_i[b]).start()
        pltpu.make_async_copy(et_hbm.at[pl.ds(e0, CH)], et_v.at[b], sem_i[b]).start()
        pltpu.make_async_copy(dst_hbm.at[pl.ds(e0, CH)], dst_v.at[b], sem_i[b]).start()

    def _idx_wait(g, b):
        e0 = ebase + g * CH
        pltpu.make_async_copy(src_hbm.at[pl.ds(e0, CH)], src_v.at[b], sem_i[b]).wait()
        pltpu.make_async_copy(et_hbm.at[pl.ds(e0, CH)], et_v.at[b], sem_i[b]).wait()
        pltpu.make_async_copy(dst_hbm.at[pl.ds(e0, CH)], dst_v.at[b], sem_i[b]).wait()

    def _keys(b):
        for o in (0, 16, CH - 16):
            sl = pl.ds(o, 16)
            key_v[b, sl] = et_v[b, sl] * N + src_v[b, sl]
            sdst_v[b, sl] = dst_v[b, sl]

    def _gather_start(b):
        pltpu.make_async_copy(t_hbm.at[key_v.at[b]], rows_v.at[b], sem_g[b]).start()

    def _gather_wait(b):
        pltpu.make_async_copy(t_hbm.at[key_v.at[b]], rows_v.at[b], sem_g[b]).wait()

    def _unpack(b, f):
        mask = jnp.full((16,), -65536, jnp.int32)
        woff = c * WH

        def _urow(j, carry):
            for k in range(WH // 16):
                w = rows_v[b, j, pl.ds(woff + k * 16, 16)]
                lo = lax.bitcast_convert_type(w << 16, jnp.float32)
                hi = lax.bitcast_convert_type(w & mask, jnp.float32)
                full_v[f, j, pl.ds(k * 16, 16)] = lo
                full_v[f, j, pl.ds(WH + k * 16, 16)] = hi
            return carry
        lax.fori_loop(0, CH, _urow, 0)

    def _scat_start(b, f):
        pltpu.make_async_copy(full_v.at[f], accum.at[sdst_v.at[b]],
                              sem_s[f]).start(add=True)

    def _scat_wait(b, f):
        pltpu.make_async_copy(full_v.at[f], accum.at[sdst_v.at[b]],
                              sem_s[f]).wait()

    for b in range(NBUF):
        _idx_start(b, b)
    for g in range(2):
        _idx_wait(g, g)
        _keys(g)
        _gather_start(g)
        _idx_start(g + NBUF, g)
    for g in (2, 3):
        _gather_wait(g - 2)
        _unpack(g - 2, (g - 2) % NFB)
        _scat_start(g - 2, (g - 2) % NFB)
        _idx_wait(g, g)
        _keys(g)
        _gather_start(g)
        _idx_start(g + NBUF, g)

    def _quad(p, carry):
        g0 = NBUF + NBUF * p
        for j in range(NBUF):
            g = g0 + j
            _gather_wait((j + 2) % NBUF)
            _scat_wait(j, j % NFB)
            _unpack((j + 2) % NBUF, j % NFB)
            _scat_start((j + 2) % NBUF, j % NFB)
            _idx_wait(g, j)
            _keys(j)
            _gather_start(j)

            @pl.when(g + NBUF < NCH)
            def _():
                _idx_start(g + NBUF, j)
        return carry

    lax.fori_loop(0, (NCH - NBUF - 2) // NBUF, _quad, 0)

    for g, b in ((NCH - 2, 0), (NCH - 1, 1)):
        _gather_wait((b + 2) % NBUF)
        _scat_wait(b, b % NFB)
        _unpack((b + 2) % NBUF, b % NFB)
        _scat_start((b + 2) % NBUF, b % NFB)
        _idx_wait(g, b)
        _keys(b)
        _gather_start(b)
    _gather_wait(0)
    _scat_wait(2, 0)
    _unpack(0, 0)
    _scat_start(0, 0)
    _gather_wait(1)
    _scat_wait(3, 1)
    _unpack(1, 1)
    _scat_start(1, 1)
    _scat_wait(0, 0)
    _scat_wait(1, 1)
    plsc.subcore_barrier()

    def _drain(i, carry):
        r0 = (s + i * NT) * ZR
        pltpu.sync_copy(accum.at[pl.ds(r0, ZR)], full_v.at[0])
        pltpu.sync_copy(full_v.at[0], out_hbm.at[pl.ds(r0, ZR), pl.ds(c * H, H)])
        return carry
    lax.fori_loop(0, nzc, _drain, 0)


def _aggregate(t, src, et, dst):
    mesh = plsc.VectorSubcoreMesh(core_axis_name="c", subcore_axis_name="s")
    f = pl.kernel(
        _sc_body,
        mesh=mesh,
        out_type=jax.ShapeDtypeStruct((N, D), jnp.float32),
        scratch_types=[
            pltpu.VMEM_SHARED((N, H), jnp.float32),
            pltpu.VMEM((NBUF, CH), jnp.int32),
            pltpu.VMEM((NBUF, CH), jnp.int32),
            pltpu.VMEM((NBUF, CH), jnp.int32),
            pltpu.VMEM((NBUF, CH), jnp.int32),
            pltpu.VMEM((NBUF, CH), jnp.int32),
            pltpu.VMEM((NBUF, CH, WPR), jnp.int32),
            pltpu.VMEM((NFB, CH, H), jnp.float32),
        ] + [pltpu.SemaphoreType.DMA] * (2 * NBUF + NFB),
    )
    return f(t, src, et, dst)


def kernel(feat, edge_index, etypes, weight):
    t = _transform(feat, weight)
    return _aggregate(t, edge_index[0], etypes, edge_index[1])


# speedup vs baseline: 1.2674x; 1.2674x over previous
import functools

import jax
import jax.numpy as jnp
from jax import lax
from jax.experimental import pallas as pl
from jax.experimental.pallas import tpu as pltpu
from jax.experimental.pallas import tpu_sc as plsc

N = 10000
E = 160000
D = 256
R = 8
H = 128
NC = 2
NT = 16
CH = 40
EPT = E // (NC * NT)
NCH = EPT // CH
NBUF = 4
BN = 1000
NB = N // BN
ZR = 40
NZ = N // ZR


def _mm_body(feat_ref, w_ref, t_ref):
    t_ref[...] = jnp.dot(feat_ref[...], w_ref[0],
                         preferred_element_type=jnp.float32)


def _transform_half(feat, weight, c):
    return pl.pallas_call(
        _mm_body,
        grid=(NB, R),
        in_specs=[
            pl.BlockSpec((BN, D), lambda i, r: (i, 0)),
            pl.BlockSpec((1, D, H), lambda i, r, c=c: (r, 0, c)),
        ],
        out_specs=pl.BlockSpec((BN, H), lambda i, r: (r * NB + i, 0)),
        out_shape=jax.ShapeDtypeStruct((R * N, H), jnp.float32),
    )(feat, weight)


def _sc_body(t_hbm, src_hbm, et_hbm, dst_hbm, out_hbm,
             accum, src_v, et_v, dst_v, key_v, sdst_v, rows_v,
             *sems):
    c = lax.axis_index("c")
    s = lax.axis_index("s")
    sem_i = sems[0:NBUF]
    sem_g = sems[NBUF:2 * NBUF]
    sem_s = sems[2 * NBUF:3 * NBUF]

    def _zrow(j, carry):
        for k in range(H // 16):
            rows_v[0, j, pl.ds(k * 16, 16)] = jnp.zeros((16,), jnp.float32)
        return carry
    lax.fori_loop(0, ZR, _zrow, 0)
    nzc = (NZ - s + NT - 1) // NT

    def _zchunk(i, carry):
        pltpu.sync_copy(rows_v.at[0], accum.at[pl.ds((s + i * NT) * ZR, ZR)])
        return carry
    lax.fori_loop(0, nzc, _zchunk, 0)
    plsc.subcore_barrier()

    ebase = (c * NT + s) * EPT

    def _idx_start(g, b):
        e0 = ebase + g * CH
        pltpu.make_async_copy(src_hbm.at[pl.ds(e0, CH)], src_v.at[b], sem_i[b]).start()
        pltpu.make_async_copy(et_hbm.at[pl.ds(e0, CH)], et_v.at[b], sem_i[b]).start()
        pltpu.make_async_copy(dst_hbm.at[pl.ds(e0, CH)], dst_v.at[b], sem_i[b]).start()

    def _idx_wait(g, b):
        e0 = ebase + g * CH
        pltpu.make_async_copy(src_hbm.at[pl.ds(e0, CH)], src_v.at[b], sem_i[b]).wait()
        pltpu.make_async_copy(et_hbm.at[pl.ds(e0, CH)], et_v.at[b], sem_i[b]).wait()
        pltpu.make_async_copy(dst_hbm.at[pl.ds(e0, CH)], dst_v.at[b], sem_i[b]).wait()

    def _keys(b):
        for o in (0, 16, CH - 16):
            sl = pl.ds(o, 16)
            key_v[b, sl] = et_v[b, sl] * N + src_v[b, sl]
            sdst_v[b, sl] = dst_v[b, sl]

    def _gather_start(b):
        pltpu.make_async_copy(t_hbm.at[key_v.at[b]], rows_v.at[b], sem_g[b]).start()

    def _gather_wait(b):
        pltpu.make_async_copy(t_hbm.at[key_v.at[b]], rows_v.at[b], sem_g[b]).wait()

    def _scat_start(b):
        pltpu.make_async_copy(rows_v.at[b], accum.at[sdst_v.at[b]],
                              sem_s[b]).start(add=True)

    def _scat_wait(b):
        pltpu.make_async_copy(rows_v.at[b], accum.at[sdst_v.at[b]],
                              sem_s[b]).wait()

    for b in range(NBUF):
        _idx_start(b, b)
    for g in range(2):
        _idx_wait(g, g)
        _keys(g)
        _gather_start(g)
        _idx_start(g + NBUF, g)
    for g in (2, 3):
        _gather_wait(g - 2)
        _scat_start(g - 2)
        _idx_wait(g, g)
        _keys(g)
        _gather_start(g)
        _idx_start(g + NBUF, g)

    def _quad(p, carry):
        g0 = NBUF + NBUF * p
        for j in range(NBUF):
            g = g0 + j
            _gather_wait((j + 2) % NBUF)
            _scat_start((j + 2) % NBUF)
            _idx_wait(g, j)
            _scat_wait(j)
            _keys(j)
            _gather_start(j)

            @pl.when(g + NBUF < NCH)
            def _():
                _idx_start(g + NBUF, j)
        return carry

    lax.fori_loop(0, (NCH - NBUF - 1) // NBUF, _quad, 0)

    _gather_wait(2)
    _scat_start(2)
    _idx_wait(NCH - 1, 0)
    _scat_wait(0)
    _keys(0)
    _gather_start(0)
    _gather_wait(3)
    _scat_start(3)
    _gather_wait(0)
    _scat_start(0)
    for b in (1, 2, 3, 0):
        _scat_wait(b)
    plsc.subcore_barrier()

    def _drain(i, carry):
        r0 = (s + i * NT) * ZR
        pltpu.sync_copy(accum.at[pl.ds(r0, ZR)], rows_v.at[0])
        pltpu.sync_copy(rows_v.at[0], out_hbm.at[c].at[pl.ds(r0, ZR)])
        return carry
    lax.fori_loop(0, nzc, _drain, 0)


def _aggregate_half(t, src, et, dst):
    mesh = plsc.VectorSubcoreMesh(core_axis_name="c", subcore_axis_name="s")
    f = pl.kernel(
        _sc_body,
        mesh=mesh,
        out_type=jax.ShapeDtypeStruct((NC, N, H), jnp.float32),
        scratch_types=[
            pltpu.VMEM_SHARED((N, H), jnp.float32),
            pltpu.VMEM((NBUF, CH), jnp.int32),
            pltpu.VMEM((NBUF, CH), jnp.int32),
            pltpu.VMEM((NBUF, CH), jnp.int32),
            pltpu.VMEM((NBUF, CH), jnp.int32),
            pltpu.VMEM((NBUF, CH), jnp.int32),
            pltpu.VMEM((NBUF, CH, H), jnp.float32),
        ] + [pltpu.SemaphoreType.DMA] * (3 * NBUF),
    )
    return f(t, src, et, dst)


def _cmb_body(p0_ref, p1_ref, out_ref):
    out_ref[:, 0:H] = p0_ref[0] + p0_ref[1]
    out_ref[:, H:D] = p1_ref[0] + p1_ref[1]


def _combine(p0, p1):
    return pl.pallas_call(
        _cmb_body,
        grid=(NB,),
        in_specs=[
            pl.BlockSpec((NC, BN, H), lambda i: (0, i, 0)),
            pl.BlockSpec((NC, BN, H), lambda i: (0, i, 0)),
        ],
        out_specs=pl.BlockSpec((BN, D), lambda i: (i, 0)),
        out_shape=jax.ShapeDtypeStruct((N, D), jnp.float32),
    )(p0, p1)


def kernel(feat, edge_index, etypes, weight):
    src, dst = edge_index[0], edge_index[1]
    t0 = _transform_half(feat, weight, 0)
    p0 = _aggregate_half(t0, src, etypes, dst)
    t1 = _transform_half(feat, weight, 1)
    p1 = _aggregate_half(t1, src, etypes, dst)
    return _combine(p0, p1)


# speedup vs baseline: 1.3916x; 1.0979x over previous
import functools

import jax
import jax.numpy as jnp
from jax import lax
from jax.experimental import pallas as pl
from jax.experimental.pallas import tpu as pltpu
from jax.experimental.pallas import tpu_sc as plsc

N = 10000
E = 160000
D = 256
R = 8
H = 128
NC = 2
NT = 16
CH = 40
EPT = E // (NC * NT)
NCH = EPT // CH
NBUF = 5
BN = 2000
NB = N // BN
ZR = 40
NZ = N // ZR


def _mm_body(feat_ref, w_ref, t_ref):
    t_ref[...] = jnp.dot(feat_ref[...], w_ref[0],
                         preferred_element_type=jnp.float32)


def _transform_half(feat, weight, c):
    return pl.pallas_call(
        _mm_body,
        grid=(NB, R),
        in_specs=[
            pl.BlockSpec((BN, D), lambda i, r: (i, 0)),
            pl.BlockSpec((1, D, H), lambda i, r, c=c: (r, 0, c)),
        ],
        out_specs=pl.BlockSpec((BN, H), lambda i, r: (r * NB + i, 0)),
        out_shape=jax.ShapeDtypeStruct((R * N, H), jnp.float32),
    )(feat, weight)


def _sc_body(t_hbm, src_hbm, et_hbm, dst_hbm, out_hbm,
             accum, src_v, et_v, dst_v, key_v, sdst_v, rows_v,
             *sems):
    c = lax.axis_index("c")
    s = lax.axis_index("s")
    sem_i = sems[0:NBUF]
    sem_g = sems[NBUF:2 * NBUF]
    sem_s = sems[2 * NBUF:3 * NBUF]

    def _zrow(j, carry):
        for k in range(H // 16):
            rows_v[0, j, pl.ds(k * 16, 16)] = jnp.zeros((16,), jnp.float32)
        return carry
    lax.fori_loop(0, ZR, _zrow, 0)
    nzc = (NZ - s + NT - 1) // NT

    def _zchunk(i, carry):
        pltpu.sync_copy(rows_v.at[0], accum.at[pl.ds((s + i * NT) * ZR, ZR)])
        return carry
    lax.fori_loop(0, nzc, _zchunk, 0)
    plsc.subcore_barrier()

    ebase = (c * NT + s) * EPT

    def _idx_start(g, b):
        e0 = ebase + g * CH
        pltpu.make_async_copy(src_hbm.at[pl.ds(e0, CH)], src_v.at[b], sem_i[b]).start()
        pltpu.make_async_copy(et_hbm.at[pl.ds(e0, CH)], et_v.at[b], sem_i[b]).start()
        pltpu.make_async_copy(dst_hbm.at[pl.ds(e0, CH)], dst_v.at[b], sem_i[b]).start()

    def _idx_wait(g, b):
        e0 = ebase + g * CH
        pltpu.make_async_copy(src_hbm.at[pl.ds(e0, CH)], src_v.at[b], sem_i[b]).wait()
        pltpu.make_async_copy(et_hbm.at[pl.ds(e0, CH)], et_v.at[b], sem_i[b]).wait()
        pltpu.make_async_copy(dst_hbm.at[pl.ds(e0, CH)], dst_v.at[b], sem_i[b]).wait()

    def _keys(b):
        for o in (0, 16, CH - 16):
            sl = pl.ds(o, 16)
            key_v[b, sl] = et_v[b, sl] * N + src_v[b, sl]
            sdst_v[b, sl] = dst_v[b, sl]

    def _gather_start(b):
        pltpu.make_async_copy(t_hbm.at[key_v.at[b]], rows_v.at[b], sem_g[b]).start()

    def _gather_wait(b):
        pltpu.make_async_copy(t_hbm.at[key_v.at[b]], rows_v.at[b], sem_g[b]).wait()

    def _scat_start(b):
        pltpu.make_async_copy(rows_v.at[b], accum.at[sdst_v.at[b]],
                              sem_s[b]).start(add=True)

    def _scat_wait(b):
        pltpu.make_async_copy(rows_v.at[b], accum.at[sdst_v.at[b]],
                              sem_s[b]).wait()

    for b in range(NBUF):
        _idx_start(b, b)
    for g in range(2):
        _idx_wait(g, g)
        _keys(g)
        _gather_start(g)
        _idx_start(g + NBUF, g)
    for g in (2, 3, 4):
        _gather_wait(g - 2)
        _scat_start(g - 2)
        _idx_wait(g, g)
        _keys(g)
        _gather_start(g)
        _idx_start(g + NBUF, g)

    def _group(p, carry):
        g0 = NBUF + NBUF * p
        for j in range(NBUF):
            g = g0 + j
            _gather_wait((j + NBUF - 2) % NBUF)
            _scat_start((j + NBUF - 2) % NBUF)
            _idx_wait(g, j)
            _scat_wait(j)
            _keys(j)
            _gather_start(j)

            @pl.when(g + NBUF < NCH)
            def _():
                _idx_start(g + NBUF, j)
        return carry

    lax.fori_loop(0, (NCH - NBUF) // NBUF, _group, 0)

    _gather_wait((NCH - 2) % NBUF)
    _scat_start((NCH - 2) % NBUF)
    _gather_wait((NCH - 1) % NBUF)
    _scat_start((NCH - 1) % NBUF)
    for g in range(NCH - NBUF, NCH):
        _scat_wait(g % NBUF)
    plsc.subcore_barrier()

    def _drain(i, carry):
        r0 = (s + i * NT) * ZR
        pltpu.sync_copy(accum.at[pl.ds(r0, ZR)], rows_v.at[0])
        pltpu.sync_copy(rows_v.at[0], out_hbm.at[c].at[pl.ds(r0, ZR)])
        return carry
    lax.fori_loop(0, nzc, _drain, 0)


def _aggregate_half(t, src, et, dst):
    mesh = plsc.VectorSubcoreMesh(core_axis_name="c", subcore_axis_name="s")
    f = pl.kernel(
        _sc_body,
        mesh=mesh,
        out_type=jax.ShapeDtypeStruct((NC, N, H), jnp.float32),
        scratch_types=[
            pltpu.VMEM_SHARED((N, H), jnp.float32),
            pltpu.VMEM((NBUF, CH), jnp.int32),
            pltpu.VMEM((NBUF, CH), jnp.int32),
            pltpu.VMEM((NBUF, CH), jnp.int32),
            pltpu.VMEM((NBUF, CH), jnp.int32),
            pltpu.VMEM((NBUF, CH), jnp.int32),
            pltpu.VMEM((NBUF, CH, H), jnp.float32),
        ] + [pltpu.SemaphoreType.DMA] * (3 * NBUF),
    )
    return f(t, src, et, dst)


def _cmb_body(p0_ref, p1_ref, out_ref):
    out_ref[:, 0:H] = p0_ref[0] + p0_ref[1]
    out_ref[:, H:D] = p1_ref[0] + p1_ref[1]


def _combine(p0, p1):
    return pl.pallas_call(
        _cmb_body,
        grid=(NB,),
        in_specs=[
            pl.BlockSpec((NC, BN, H), lambda i: (0, i, 0)),
            pl.BlockSpec((NC, BN, H), lambda i: (0, i, 0)),
        ],
        out_specs=pl.BlockSpec((BN, D), lambda i: (i, 0)),
        out_shape=jax.ShapeDtypeStruct((N, D), jnp.float32),
    )(p0, p1)


def kernel(feat, edge_index, etypes, weight):
    src, dst = edge_index[0], edge_index[1]
    t0 = _transform_half(feat, weight, 0)
    p0 = _aggregate_half(t0, src, etypes, dst)
    t1 = _transform_half(feat, weight, 1)
    p1 = _aggregate_half(t1, src, etypes, dst)
    return _combine(p0, p1)


# speedup vs baseline: 1.3962x; 1.0033x over previous
import functools

import jax
import jax.numpy as jnp
from jax import lax
from jax.experimental import pallas as pl
from jax.experimental.pallas import tpu as pltpu
from jax.experimental.pallas import tpu_sc as plsc

N = 10000
E = 160000
D = 256
R = 8
H = 128
NC = 2
NT = 16
CH = 40
EPT = E // (NC * NT)
NCH = EPT // CH
NBUF = 5
BN = 2000
NB = N // BN
ZR = 40
NZ = N // ZR


def _mm_body(feat_ref, w_ref, t_ref):
    t_ref[...] = jnp.dot(feat_ref[...], w_ref[0],
                         preferred_element_type=jnp.float32)


def _transform_half(feat, weight, c):
    return pl.pallas_call(
        _mm_body,
        grid=(NB, R),
        in_specs=[
            pl.BlockSpec((BN, D), lambda i, r: (i, 0)),
            pl.BlockSpec((1, D, H), lambda i, r, c=c: (r, 0, c)),
        ],
        out_specs=pl.BlockSpec((BN, H), lambda i, r: (r * NB + i, 0)),
        out_shape=jax.ShapeDtypeStruct((R * N, H), jnp.float32),
    )(feat, weight)


def _sc_body(t_hbm, src_hbm, et_hbm, dst_hbm, out_hbm,
             accum, src_v, et_v, dst_v, key_v, sdst_v, rows_v,
             *sems):
    c = lax.axis_index("c")
    s = lax.axis_index("s")
    sem_i = sems[0]
    sem_g = sems[1:1 + NBUF]
    sem_s = sems[1 + NBUF:1 + 2 * NBUF]

    ebase = (c * NT + s) * EPT
    esl = pl.ds(ebase, EPT)
    pltpu.make_async_copy(src_hbm.at[esl], src_v, sem_i).start()
    pltpu.make_async_copy(et_hbm.at[esl], et_v, sem_i).start()
    pltpu.make_async_copy(dst_hbm.at[esl], dst_v, sem_i).start()

    def _zrow(j, carry):
        for k in range(H // 16):
            rows_v[0, j, pl.ds(k * 16, 16)] = jnp.zeros((16,), jnp.float32)
        return carry
    lax.fori_loop(0, ZR, _zrow, 0)
    nzc = (NZ - s + NT - 1) // NT

    def _zchunk(i, carry):
        pltpu.sync_copy(rows_v.at[0], accum.at[pl.ds((s + i * NT) * ZR, ZR)])
        return carry
    lax.fori_loop(0, nzc, _zchunk, 0)
    pltpu.make_async_copy(src_hbm.at[esl], src_v, sem_i).wait()
    pltpu.make_async_copy(et_hbm.at[esl], et_v, sem_i).wait()
    pltpu.make_async_copy(dst_hbm.at[esl], dst_v, sem_i).wait()
    plsc.subcore_barrier()

    def _keys(g, b):
        for o in (0, 16, CH - 16):
            sl = pl.ds(o, 16)
            el = pl.ds(g * CH + o, 16)
            key_v[b, sl] = et_v[el] * N + src_v[el]
            sdst_v[b, sl] = dst_v[el]

    def _gather_start(b):
        pltpu.make_async_copy(t_hbm.at[key_v.at[b]], rows_v.at[b], sem_g[b]).start()

    def _gather_wait(b):
        pltpu.make_async_copy(t_hbm.at[key_v.at[b]], rows_v.at[b], sem_g[b]).wait()

    def _scat_start(b):
        pltpu.make_async_copy(rows_v.at[b], accum.at[sdst_v.at[b]],
                              sem_s[b]).start(add=True)

    def _scat_wait(b):
        pltpu.make_async_copy(rows_v.at[b], accum.at[sdst_v.at[b]],
                              sem_s[b]).wait()

    for g in range(2):
        _keys(g, g)
        _gather_start(g)
    for g in (2, 3, 4):
        _gather_wait(g - 2)
        _scat_start(g - 2)
        _keys(g, g)
        _gather_start(g)

    def _group(p, carry):
        g0 = NBUF + NBUF * p
        for j in range(NBUF):
            g = g0 + j
            _gather_wait((j + NBUF - 2) % NBUF)
            _scat_start((j + NBUF - 2) % NBUF)
            _scat_wait(j)
            _keys(g, j)
            _gather_start(j)
        return carry

    lax.fori_loop(0, (NCH - NBUF) // NBUF, _group, 0)

    _gather_wait((NCH - 2) % NBUF)
    _scat_start((NCH - 2) % NBUF)
    _gather_wait((NCH - 1) % NBUF)
    _scat_start((NCH - 1) % NBUF)
    for g in range(NCH - NBUF, NCH):
        _scat_wait(g % NBUF)
    plsc.subcore_barrier()

    def _drain(i, carry):
        r0 = (s + i * NT) * ZR
        pltpu.sync_copy(accum.at[pl.ds(r0, ZR)], rows_v.at[0])
        pltpu.sync_copy(rows_v.at[0], out_hbm.at[c].at[pl.ds(r0, ZR)])
        return carry
    lax.fori_loop(0, nzc, _drain, 0)


def _aggregate_half(t, src, et, dst):
    mesh = plsc.VectorSubcoreMesh(core_axis_name="c", subcore_axis_name="s")
    f = pl.kernel(
        _sc_body,
        mesh=mesh,
        out_type=jax.ShapeDtypeStruct((NC, N, H), jnp.float32),
        scratch_types=[
            pltpu.VMEM_SHARED((N, H), jnp.float32),
            pltpu.VMEM((EPT,), jnp.int32),
            pltpu.VMEM((EPT,), jnp.int32),
            pltpu.VMEM((EPT,), jnp.int32),
            pltpu.VMEM((NBUF, CH), jnp.int32),
            pltpu.VMEM((NBUF, CH), jnp.int32),
            pltpu.VMEM((NBUF, CH, H), jnp.float32),
        ] + [pltpu.SemaphoreType.DMA] * (1 + 2 * NBUF),
    )
    return f(t, src, et, dst)


def _cmb_body(p0_ref, p1_ref, out_ref):
    out_ref[:, 0:H] = p0_ref[0] + p0_ref[1]
    out_ref[:, H:D] = p1_ref[0] + p1_ref[1]


def _combine(p0, p1):
    return pl.pallas_call(
        _cmb_body,
        grid=(NB,),
        in_specs=[
            pl.BlockSpec((NC, BN, H), lambda i: (0, i, 0)),
            pl.BlockSpec((NC, BN, H), lambda i: (0, i, 0)),
        ],
        out_specs=pl.BlockSpec((BN, D), lambda i: (i, 0)),
        out_shape=jax.ShapeDtypeStruct((N, D), jnp.float32),
    )(p0, p1)


def kernel(feat, edge_index, etypes, weight):
    src, dst = edge_index[0], edge_index[1]
    t0 = _transform_half(feat, weight, 0)
    p0 = _aggregate_half(t0, src, etypes, dst)
    t1 = _transform_half(feat, weight, 1)
    p1 = _aggregate_half(t1, src, etypes, dst)
    return _combine(p0, p1)


# speedup vs baseline: 1.6096x; 1.1528x over previous
import functools

import jax
import jax.numpy as jnp
from jax import lax
from jax.experimental import pallas as pl
from jax.experimental.pallas import tpu as pltpu
from jax.experimental.pallas import tpu_sc as plsc

N = 10000
E = 160000
D = 256
R = 8
H = 128
NC = 2
NT = 16
CH = 40
EPT = E // (NC * NT)
NCH = EPT // CH
NBUF = 5
BN = 2000
NB = N // BN
ZR = 40
NZ = N // ZR


def _mm_body(feat_ref, w_ref, t_ref):
    t_ref[...] = jnp.dot(feat_ref[...], w_ref[0],
                         preferred_element_type=jnp.float32)


def _transform_half(feat, weight, c):
    return pl.pallas_call(
        _mm_body,
        grid=(NB, R),
        in_specs=[
            pl.BlockSpec((BN, D), lambda i, r: (i, 0)),
            pl.BlockSpec((1, D, H), lambda i, r, c=c: (r, 0, c)),
        ],
        out_specs=pl.BlockSpec((BN, H), lambda i, r: (r * NB + i, 0)),
        out_shape=jax.ShapeDtypeStruct((R * N, H), jnp.float32),
    )(feat, weight)


def _sc_body(t_hbm, src_hbm, et_hbm, dst_hbm, out_hbm,
             accum, src_v, et_v, dst_v, key_v, sdst_v, rows_v,
             *sems):
    c = lax.axis_index("c")
    s = lax.axis_index("s")
    sem_i = sems[0]
    sem_g = sems[1:1 + NBUF]
    sem_s = sems[1 + NBUF:1 + 2 * NBUF]

    ebase = (c * NT + s) * EPT
    esl = pl.ds(ebase, EPT)
    pltpu.make_async_copy(src_hbm.at[esl], src_v, sem_i).start()
    pltpu.make_async_copy(et_hbm.at[esl], et_v, sem_i).start()
    pltpu.make_async_copy(dst_hbm.at[esl], dst_v, sem_i).start()

    def _zrow(j, carry):
        for k in range(H // 16):
            rows_v[0, j, pl.ds(k * 16, 16)] = jnp.zeros((16,), jnp.float32)
        return carry
    lax.fori_loop(0, ZR, _zrow, 0)
    nzc = (NZ - s + NT - 1) // NT

    def _zchunk(i, carry):
        pltpu.sync_copy(rows_v.at[0], accum.at[pl.ds((s + i * NT) * ZR, ZR)])
        return carry
    lax.fori_loop(0, nzc, _zchunk, 0)
    pltpu.make_async_copy(src_hbm.at[esl], src_v, sem_i).wait()
    pltpu.make_async_copy(et_hbm.at[esl], et_v, sem_i).wait()
    pltpu.make_async_copy(dst_hbm.at[esl], dst_v, sem_i).wait()
    plsc.subcore_barrier()

    def _keys(g, b):
        for o in (0, 16, CH - 16):
            sl = pl.ds(o, 16)
            el = pl.ds(g * CH + o, 16)
            key_v[b, sl] = et_v[el] * N + src_v[el]
            sdst_v[b, sl] = dst_v[el]

    def _gather_start(b):
        pltpu.make_async_copy(t_hbm.at[key_v.at[b]], rows_v.at[b], sem_g[b]).start()

    def _gather_wait(b):
        pltpu.make_async_copy(t_hbm.at[key_v.at[b]], rows_v.at[b], sem_g[b]).wait()

    def _scat_start(b):
        pltpu.make_async_copy(rows_v.at[b], accum.at[sdst_v.at[b]],
                              sem_s[b]).start(add=True)

    def _scat_wait(b):
        pltpu.make_async_copy(rows_v.at[b], accum.at[sdst_v.at[b]],
                              sem_s[b]).wait()

    for g in range(3):
        _keys(g, g)
        _gather_start(g)
    for g in (3, 4):
        _gather_wait(g - 3)
        _scat_start(g - 3)
        _keys(g, g)
        _gather_start(g)

    def _group(p, carry):
        g0 = NBUF + NBUF * p
        for j in range(NBUF):
            g = g0 + j
            _gather_wait((j + NBUF - 3) % NBUF)
            _scat_start((j + NBUF - 3) % NBUF)
            _scat_wait(j)
            _keys(g, j)
            _gather_start(j)
        return carry

    lax.fori_loop(0, (NCH - NBUF) // NBUF, _group, 0)

    for g in (NCH - 3, NCH - 2, NCH - 1):
        _gather_wait(g % NBUF)
        _scat_start(g % NBUF)
    for g in range(NCH - NBUF, NCH):
        _scat_wait(g % NBUF)
    plsc.subcore_barrier()

    def _drain(i, carry):
        r0 = (s + i * NT) * ZR
        pltpu.sync_copy(accum.at[pl.ds(r0, ZR)], rows_v.at[0])
        pltpu.sync_copy(rows_v.at[0], out_hbm.at[c].at[pl.ds(r0, ZR)])
        return carry
    lax.fori_loop(0, nzc, _drain, 0)


def _aggregate_half(t, src, et, dst):
    mesh = plsc.VectorSubcoreMesh(core_axis_name="c", subcore_axis_name="s")
    f = pl.kernel(
        _sc_body,
        mesh=mesh,
        out_type=jax.ShapeDtypeStruct((NC, N, H), jnp.float32),
        scratch_types=[
            pltpu.VMEM_SHARED((N, H), jnp.float32),
            pltpu.VMEM((EPT,), jnp.int32),
            pltpu.VMEM((EPT,), jnp.int32),
            pltpu.VMEM((EPT,), jnp.int32),
            pltpu.VMEM((NBUF, CH), jnp.int32),
            pltpu.VMEM((NBUF, CH), jnp.int32),
            pltpu.VMEM((NBUF, CH, H), jnp.float32),
        ] + [pltpu.SemaphoreType.DMA] * (1 + 2 * NBUF),
    )
    return f(t, src, et, dst)


def _cmb_body(p0_ref, p1_ref, out_ref):
    out_ref[:, 0:H] = p0_ref[0] + p0_ref[1]
    out_ref[:, H:D] = p1_ref[0] + p1_ref[1]


def _combine(p0, p1):
    return pl.pallas_call(
        _cmb_body,
        grid=(NB,),
        in_specs=[
            pl.BlockSpec((NC, BN, H), lambda i: (0, i, 0)),
            pl.BlockSpec((NC, BN, H), lambda i: (0, i, 0)),
        ],
        out_specs=pl.BlockSpec((BN, D), lambda i: (i, 0)),
        out_shape=jax.ShapeDtypeStruct((N, D), jnp.float32),
    )(p0, p1)


def kernel(feat, edge_index, etypes, weight):
    src, dst = edge_index[0], edge_index[1]
    t0 = _transform_half(feat, weight, 0)
    p0 = _aggregate_half(t0, src, etypes, dst)
    t1 = _transform_half(feat, weight, 1)
    p1 = _aggregate_half(t1, src, etypes, dst)
    return _combine(p0, p1)


# speedup vs baseline: 1.7039x; 1.0586x over previous
import functools

import jax
import jax.numpy as jnp
from jax import lax
from jax.experimental import pallas as pl
from jax.experimental.pallas import tpu as pltpu
from jax.experimental.pallas import tpu_sc as plsc

N = 10000
E = 160000
D = 256
R = 8
H = 128
NC = 2
NT = 16
CH = 40
EPT = E // (NC * NT)
NCH = EPT // CH
NBUF = 5
BN = 2000
NB = N // BN
ZR = 40
NZ = N // ZR


def _mm_body(feat_ref, w_ref, t_ref):
    t_ref[...] = jnp.dot(feat_ref[...], w_ref[0],
                         preferred_element_type=jnp.float32)


def _transform_half(feat, weight, c):
    return pl.pallas_call(
        _mm_body,
        grid=(NB, R),
        in_specs=[
            pl.BlockSpec((BN, D), lambda i, r: (i, 0)),
            pl.BlockSpec((1, D, H), lambda i, r, c=c: (r, 0, c)),
        ],
        out_specs=pl.BlockSpec((BN, H), lambda i, r: (r * NB + i, 0)),
        out_shape=jax.ShapeDtypeStruct((R * N, H), jnp.float32),
    )(feat, weight)


def _sc_body(t_hbm, src_hbm, et_hbm, dst_hbm, out_hbm,
             accum, src_v, et_v, dst_v, key_v, sdst_v, rows_v,
             *sems):
    c = lax.axis_index("c")
    s = lax.axis_index("s")
    sem_i = sems[0]
    sem_g = sems[1:1 + NBUF]
    sem_s = sems[1 + NBUF:1 + 2 * NBUF]

    ebase = (c * NT + s) * EPT
    esl = pl.ds(ebase, EPT)
    pltpu.make_async_copy(src_hbm.at[esl], src_v, sem_i).start()
    pltpu.make_async_copy(et_hbm.at[esl], et_v, sem_i).start()
    pltpu.make_async_copy(dst_hbm.at[esl], dst_v, sem_i).start()

    def _zrow(j, carry):
        for k in range(H // 16):
            rows_v[0, j, pl.ds(k * 16, 16)] = jnp.zeros((16,), jnp.float32)
        return carry
    lax.fori_loop(0, ZR, _zrow, 0)
    nzc = (NZ - s + NT - 1) // NT

    def _zchunk(i, carry):
        pltpu.sync_copy(rows_v.at[0], accum.at[pl.ds((s + i * NT) * ZR, ZR)])
        return carry
    lax.fori_loop(0, nzc, _zchunk, 0)
    pltpu.make_async_copy(src_hbm.at[esl], src_v, sem_i).wait()
    pltpu.make_async_copy(et_hbm.at[esl], et_v, sem_i).wait()
    pltpu.make_async_copy(dst_hbm.at[esl], dst_v, sem_i).wait()
    plsc.subcore_barrier()

    def _keys(g, b):
        for o in (0, 16, CH - 16):
            sl = pl.ds(o, 16)
            el = pl.ds(g * CH + o, 16)
            key_v[b, sl] = et_v[el] * N + src_v[el]
            sdst_v[b, sl] = dst_v[el]

    def _gather_start(b):
        pltpu.make_async_copy(t_hbm.at[key_v.at[b]], rows_v.at[b], sem_g[b]).start()

    def _gather_wait(b):
        pltpu.make_async_copy(t_hbm.at[key_v.at[b]], rows_v.at[b], sem_g[b]).wait()

    def _scat_start(b):
        pltpu.make_async_copy(rows_v.at[b], accum.at[sdst_v.at[b]],
                              sem_s[b]).start(add=True)

    def _scat_wait(b):
        pltpu.make_async_copy(rows_v.at[b], accum.at[sdst_v.at[b]],
                              sem_s[b]).wait()

    for g in range(4):
        _keys(g, g)
        _gather_start(g)
    _gather_wait(0)
    _scat_start(0)
    _keys(4, 4)
    _gather_start(4)

    def _group(p, carry):
        g0 = NBUF + NBUF * p
        for j in range(NBUF):
            g = g0 + j
            _gather_wait((j + NBUF - 4) % NBUF)
            _scat_start((j + NBUF - 4) % NBUF)
            _scat_wait(j)
            _keys(g, j)
            _gather_start(j)
        return carry

    lax.fori_loop(0, (NCH - NBUF) // NBUF, _group, 0)

    for g in (NCH - 4, NCH - 3, NCH - 2, NCH - 1):
        _gather_wait(g % NBUF)
        _scat_start(g % NBUF)
    for g in range(NCH - NBUF, NCH):
        _scat_wait(g % NBUF)
    plsc.subcore_barrier()

    def _drain(i, carry):
        r0 = (s + i * NT) * ZR
        pltpu.sync_copy(accum.at[pl.ds(r0, ZR)], rows_v.at[0])
        pltpu.sync_copy(rows_v.at[0], out_hbm.at[c].at[pl.ds(r0, ZR)])
        return carry
    lax.fori_loop(0, nzc, _drain, 0)


def _aggregate_half(t, src, et, dst):
    mesh = plsc.VectorSubcoreMesh(core_axis_name="c", subcore_axis_name="s")
    f = pl.kernel(
        _sc_body,
        mesh=mesh,
        out_type=jax.ShapeDtypeStruct((NC, N, H), jnp.float32),
        scratch_types=[
            pltpu.VMEM_SHARED((N, H), jnp.float32),
            pltpu.VMEM((EPT,), jnp.int32),
            pltpu.VMEM((EPT,), jnp.int32),
            pltpu.VMEM((EPT,), jnp.int32),
            pltpu.VMEM((NBUF, CH), jnp.int32),
            pltpu.VMEM((NBUF, CH), jnp.int32),
            pltpu.VMEM((NBUF, CH, H), jnp.float32),
        ] + [pltpu.SemaphoreType.DMA] * (1 + 2 * NBUF),
    )
    return f(t, src, et, dst)


def _cmb_body(p0_ref, p1_ref, out_ref):
    out_ref[:, 0:H] = p0_ref[0] + p0_ref[1]
    out_ref[:, H:D] = p1_ref[0] + p1_ref[1]


def _combine(p0, p1):
    return pl.pallas_call(
        _cmb_body,
        grid=(NB,),
        in_specs=[
            pl.BlockSpec((NC, BN, H), lambda i: (0, i, 0)),
            pl.BlockSpec((NC, BN, H), lambda i: (0, i, 0)),
        ],
        out_specs=pl.BlockSpec((BN, D), lambda i: (i, 0)),
        out_shape=jax.ShapeDtypeStruct((N, D), jnp.float32),
    )(p0, p1)


def kernel(feat, edge_index, etypes, weight):
    src, dst = edge_index[0], edge_index[1]
    t0 = _transform_half(feat, weight, 0)
    p0 = _aggregate_half(t0, src, etypes, dst)
    t1 = _transform_half(feat, weight, 1)
    p1 = _aggregate_half(t1, src, etypes, dst)
    return _combine(p0, p1)
